# baseline (speedup 1.0000x reference)
import functools

import jax
import jax.numpy as jnp
from jax import lax
from jax.experimental import pallas as pl
from jax.experimental.pallas import tpu as pltpu
from jax.experimental.pallas import tpu_sc as plsc

BATCH = 4
SEQ = 8192
D_MODEL = 1024
N_IDX = BATCH * SEQ

_info = plsc.get_sparse_core_info()
_NC, _NS = _info.num_cores, _info.num_subcores
_NW = _NC * _NS
_B_PER_W = N_IDX // _NW
_W_PER_ROW = SEQ // _B_PER_W
_CHUNK = 16
_N_CHUNKS = _B_PER_W // _CHUNK
_NBUF = 6
_LOOKAHEAD = 3


@functools.partial(
    pl.kernel,
    mesh=plsc.VectorSubcoreMesh(core_axis_name="c", subcore_axis_name="s"),
    out_type=jax.ShapeDtypeStruct((BATCH, SEQ, D_MODEL), jnp.float32),
    scratch_types=[
        pltpu.VMEM((_B_PER_W,), jnp.int32),
        pltpu.VMEM((_NBUF, _CHUNK, D_MODEL), jnp.float32),
        pltpu.SemaphoreType.DMA,
        pltpu.SemaphoreType.DMA,
    ],
)
def _gather_rows(pe_hbm, idx_hbm, out_hbm, idx_v, rows_v, gsem, osem):
    wid = lax.axis_index("s") * _NC + lax.axis_index("c")
    row = wid // _W_PER_ROW
    col = (wid % _W_PER_ROW) * _B_PER_W

    def gather(g, buf):
        return pltpu.make_async_copy(
            pe_hbm.at[idx_v.at[pl.ds(g * _CHUNK, _CHUNK)]], rows_v.at[buf], gsem
        )

    def out_copy(g, buf):
        return pltpu.make_async_copy(
            rows_v.at[buf], out_hbm.at[row, pl.ds(col + g * _CHUNK, _CHUNK)], osem
        )

    pltpu.sync_copy(idx_hbm.at[row, pl.ds(col, _B_PER_W)], idx_v)
    for j in range(_LOOKAHEAD):
        gather(j, j % _NBUF).start()

    def outer(i, carry):
        for b in range(_NBUF):
            g = i * _NBUF + b

            @pl.when(g < _N_CHUNKS)
            def _():
                nxt = (b + _LOOKAHEAD) % _NBUF

                @pl.when(g + _LOOKAHEAD < _N_CHUNKS)
                def _():
                    @pl.when(g + _LOOKAHEAD >= _NBUF)
                    def _():
                        out_copy(g + _LOOKAHEAD - _NBUF, nxt).wait()

                    gather(g + _LOOKAHEAD, nxt).start()

                gather(g, b).wait()
                out_copy(g, b).start()
        return carry

    lax.fori_loop(0, (_N_CHUNKS + _NBUF - 1) // _NBUF, outer, 0)
    for g in range(_N_CHUNKS - _NBUF, _N_CHUNKS):
        out_copy(g, g % _NBUF).wait()


def kernel(t, pe):
    if t.dtype != jnp.int32:
        t = t.astype(jnp.int32)
    return _gather_rows(pe, t)


# speedup vs baseline: 1.0110x; 1.0110x over previous
import functools

import jax
import jax.numpy as jnp
from jax import lax
from jax.experimental import pallas as pl
from jax.experimental.pallas import tpu as pltpu
from jax.experimental.pallas import tpu_sc as plsc

BATCH = 4
SEQ = 8192
D_MODEL = 1024
N_IDX = BATCH * SEQ

_info = plsc.get_sparse_core_info()
_NC, _NS = _info.num_cores, _info.num_subcores
_NW = _NC * _NS
_B_PER_W = N_IDX // _NW
_W_PER_ROW = SEQ // _B_PER_W
_CHUNK = 32
_N_CHUNKS = _B_PER_W // _CHUNK
_NBUF = 3
_LOOKAHEAD = 2


@functools.partial(
    pl.kernel,
    mesh=plsc.VectorSubcoreMesh(core_axis_name="c", subcore_axis_name="s"),
    out_type=jax.ShapeDtypeStruct((BATCH, SEQ, D_MODEL), jnp.float32),
    scratch_types=[
        pltpu.VMEM((_B_PER_W,), jnp.int32),
        pltpu.VMEM((_NBUF, _CHUNK, D_MODEL), jnp.float32),
        pltpu.SemaphoreType.DMA,
        pltpu.SemaphoreType.DMA,
    ],
)
def _gather_rows(pe_hbm, idx_hbm, out_hbm, idx_v, rows_v, gsem, osem):
    wid = lax.axis_index("s") * _NC + lax.axis_index("c")
    row = wid // _W_PER_ROW
    col = (wid % _W_PER_ROW) * _B_PER_W

    def gather(g, buf):
        return pltpu.make_async_copy(
            pe_hbm.at[idx_v.at[pl.ds(g * _CHUNK, _CHUNK)]], rows_v.at[buf], gsem
        )

    def out_copy(g, buf):
        return pltpu.make_async_copy(
            rows_v.at[buf], out_hbm.at[row, pl.ds(col + g * _CHUNK, _CHUNK)], osem
        )

    pltpu.sync_copy(idx_hbm.at[row, pl.ds(col, _B_PER_W)], idx_v)
    for j in range(_LOOKAHEAD):
        gather(j, j % _NBUF).start()

    def outer(i, carry):
        for b in range(_NBUF):
            g = i * _NBUF + b

            @pl.when(g < _N_CHUNKS)
            def _():
                nxt = (b + _LOOKAHEAD) % _NBUF

                @pl.when(g + _LOOKAHEAD < _N_CHUNKS)
                def _():
                    @pl.when(g + _LOOKAHEAD >= _NBUF)
                    def _():
                        out_copy(g + _LOOKAHEAD - _NBUF, nxt).wait()

                    gather(g + _LOOKAHEAD, nxt).start()

                gather(g, b).wait()
                out_copy(g, b).start()
        return carry

    lax.fori_loop(0, (_N_CHUNKS + _NBUF - 1) // _NBUF, outer, 0)
    for g in range(_N_CHUNKS - _NBUF, _N_CHUNKS):
        out_copy(g, g % _NBUF).wait()


def kernel(t, pe):
    if t.dtype != jnp.int32:
        t = t.astype(jnp.int32)
    return _gather_rows(pe, t)
